# baseline (speedup 1.0000x reference)
import functools

import jax
import jax.numpy as jnp
import numpy as np
from jax import lax
from jax.experimental import pallas as pl
from jax.experimental.pallas import tpu as pltpu
from jax.experimental.pallas import tpu_sc as plsc

H = 128
_EPS = 1e-5

_NC, _NS = 2, 16
_NW = _NC * _NS
_CHK = 80
_RPW = 125
_SUP = 400
_NSUP = _RPW * _CHK // _SUP


def _blk(n, want):
    for b in range(min(want, n), 7, -1):
        if n % b == 0 and b % 8 == 0:
            return b
    return n


def _embed_body(a_ref, b_ref, invf_ref, out_ref):
    col = lax.broadcasted_iota(jnp.int32, (1, H), 1)
    val = jnp.where(col < (H // 2), a_ref[...], b_ref[...])
    arg = val * invf_ref[...]
    out_ref[...] = jnp.where(col % 2 == 0, jnp.sin(arg), jnp.cos(arg))


def _b16(x):
    u = jax.lax.bitcast_convert_type(x, jnp.uint32)
    return (u + jnp.uint32(0x7FFF) + ((u >> 16) & jnp.uint32(1))) >> 16


def _proj_body(h_ref, w_ref, o1_ref, o2_ref, o3_ref):
    hp = jnp.dot(h_ref[...], w_ref[...], preferred_element_type=jnp.float32)
    o1_ref[...] = hp[:, 0:H]
    packed = (_b16(hp[:, H:2 * H]) << 16) | _b16(hp[:, 2 * H:3 * H])
    o2_ref[...] = jax.lax.bitcast_convert_type(packed, jnp.int32)
    o3_ref[...] = hp[:, 3 * H:4 * H]


def _unpack_rv(grv):
    pk = jax.lax.bitcast_convert_type(grv, jnp.uint32)
    gr = jax.lax.bitcast_convert_type(pk & jnp.uint32(0xFFFF0000), jnp.float32)
    gv = jax.lax.bitcast_convert_type(pk << 16, jnp.float32)
    return gr, gv


def _edge_core(e, gq, gr, P_ref, W1_ref, W2_ref, vec_ref):
    ehat = jnp.dot(e, P_ref[...], preferred_element_type=jnp.float32)
    ehat = ehat + gq + gr
    m = jnp.mean(ehat, axis=-1, keepdims=True)
    v = jnp.mean((ehat - m) ** 2, axis=-1, keepdims=True)
    u = (ehat - m) * lax.rsqrt(v + _EPS) * vec_ref[0:1, :] + vec_ref[1:2, :]
    mid = jnp.maximum(
        jnp.dot(u, W1_ref[...], preferred_element_type=jnp.float32)
        + vec_ref[2:3, :], 0.0)
    eo = (e + jnp.dot(mid, W2_ref[...], preferred_element_type=jnp.float32)
          + vec_ref[3:4, :])
    return eo, ehat


def _edge_first_body(a_ref, b_ref, invf_ref, gq_ref, grv_ref, P_ref, W1_ref,
                     W2_ref, vec_ref, eo_ref, msg_ref):
    col = lax.broadcasted_iota(jnp.int32, (1, H), 1)
    val = jnp.where(col < (H // 2), a_ref[...], b_ref[...])
    arg = val * invf_ref[...]
    e = jnp.where(col % 2 == 0, jnp.sin(arg), jnp.cos(arg))
    gr, gv = _unpack_rv(grv_ref[...])
    eo, ehat = _edge_core(e, gq_ref[...], gr, P_ref, W1_ref, W2_ref, vec_ref)
    eo_ref[...] = eo
    msg_ref[...] = jax.nn.sigmoid(ehat) * gv


def _edge_body(e_ref, gq_ref, grv_ref, P_ref, W1_ref, W2_ref,
               vec_ref, eo_ref, msg_ref):
    gr, gv = _unpack_rv(grv_ref[...])
    eo, ehat = _edge_core(e_ref[...], gq_ref[...], gr, P_ref, W1_ref, W2_ref,
                          vec_ref)
    eo_ref[...] = eo
    msg_ref[...] = jax.nn.sigmoid(ehat) * gv


def _edge_last_body(e_ref, gq_ref, grv_ref, P_ref, W1_ref, W2_ref, vec_ref,
                    hW1_ref, hb1_ref, hW2_ref, hb2_ref, out_ref):
    gr, _ = _unpack_rv(grv_ref[...])
    eo, _ = _edge_core(e_ref[...], gq_ref[...], gr, P_ref, W1_ref, W2_ref,
                       vec_ref)
    mid = jnp.maximum(
        jnp.dot(eo, hW1_ref[...], preferred_element_type=jnp.float32)
        + hb1_ref[...], 0.0)
    out_ref[...] = (jnp.dot(mid, hW2_ref[...],
                            preferred_element_type=jnp.float32) + hb2_ref[...])


def _nup_body(h_ref, hu_ref, a0_ref, a1_ref, a2_ref, a3_ref, vec_ref,
              out_ref):
    x = (hu_ref[...] + a0_ref[...] + a1_ref[...]
         + a2_ref[...] + a3_ref[...])
    m = jnp.mean(x, axis=-1, keepdims=True)
    v = jnp.mean((x - m) ** 2, axis=-1, keepdims=True)
    u = (x - m) * lax.rsqrt(v + _EPS) * vec_ref[0:1, :] + vec_ref[1:2, :]
    out_ref[...] = h_ref[...] + jnp.maximum(u, 0.0)


def _full(shape):
    return pl.BlockSpec(shape, lambda *a: (0,) * len(shape))


def _embed_call(a, b, invf):
    n = a.shape[0]
    B = _blk(n, 4000)
    grid = (n // B,)
    return pl.pallas_call(
        _embed_body,
        grid=grid,
        in_specs=[pl.BlockSpec((B, 1), lambda i: (i, 0)),
                  pl.BlockSpec((B, 1), lambda i: (i, 0)),
                  _full((1, H))],
        out_specs=pl.BlockSpec((B, H), lambda i: (i, 0)),
        out_shape=jax.ShapeDtypeStruct((n, H), jnp.float32),
    )(a, b, invf)


def _proj_call(h, w):
    n, dout = h.shape[0], w.shape[1]
    return pl.pallas_call(
        _proj_body,
        in_specs=[_full((n, H)), _full((H, dout))],
        out_specs=[_full((n, H))] * 3,
        out_shape=[jax.ShapeDtypeStruct((n, H), jnp.float32),
                   jax.ShapeDtypeStruct((n, H), jnp.int32),
                   jax.ShapeDtypeStruct((n, H), jnp.float32)],
    )(h, w)


def _edge_call(e, gq, grv, P, W1, W2, vec):
    n = e.shape[0]
    B = _blk(n, 4000)
    grid = (n // B,)
    eb = pl.BlockSpec((B, H), lambda i: (i, 0))
    return pl.pallas_call(
        _edge_body,
        grid=grid,
        in_specs=[eb, eb, eb, _full((H, H)), _full((H, H)),
                  _full((H, H)), _full((4, H))],
        out_specs=[eb, eb],
        out_shape=[jax.ShapeDtypeStruct((n, H), jnp.float32),
                   jax.ShapeDtypeStruct((n, H), jnp.float32)],
    )(e, gq, grv, P, W1, W2, vec)


def _edge_first_call(a, b, invf, gq, grv, P, W1, W2, vec):
    n = gq.shape[0]
    B = _blk(n, 4000)
    grid = (n // B,)
    eb = pl.BlockSpec((B, H), lambda i: (i, 0))
    sb = pl.BlockSpec((B, 1), lambda i: (i, 0))
    return pl.pallas_call(
        _edge_first_body,
        grid=grid,
        in_specs=[sb, sb, _full((1, H)), eb, eb, _full((H, H)),
                  _full((H, H)), _full((H, H)), _full((4, H))],
        out_specs=[eb, eb],
        out_shape=[jax.ShapeDtypeStruct((n, H), jnp.float32),
                   jax.ShapeDtypeStruct((n, H), jnp.float32)],
    )(a, b, invf, gq, grv, P, W1, W2, vec)


def _edge_last_call(e, gq, grv, P, W1, W2, vec, hW1, hb1, hW2, hb2):
    n = e.shape[0]
    B = _blk(n, 4000)
    grid = (n // B,)
    eb = pl.BlockSpec((B, H), lambda i: (i, 0))
    dout = hW2.shape[1]
    return pl.pallas_call(
        _edge_last_body,
        grid=grid,
        in_specs=[eb, eb, eb, _full((H, H)), _full((H, H)), _full((H, H)),
                  _full((4, H)), _full((H, H)), _full((1, H)),
                  _full((H, dout)), _full((1, dout))],
        out_specs=pl.BlockSpec((B, dout), lambda i: (i, 0)),
        out_shape=jax.ShapeDtypeStruct((n, dout), jnp.float32),
    )(e, gq, grv, P, W1, W2, vec, hW1, hb1, hW2, hb2)


def _nup_call(h, hu, aggs, vec):
    n = h.shape[0]
    B = _blk(n, 4000)
    grid = (n // B,)
    nb = pl.BlockSpec((B, H), lambda i: (i, 0))
    return pl.pallas_call(
        _nup_body,
        grid=grid,
        in_specs=[nb, nb, nb, nb, nb, nb, _full((2, H))],
        out_specs=nb,
        out_shape=jax.ShapeDtypeStruct((n, H), jnp.float32),
    )(h, hu, *aggs, vec)


def _gather2_body(chk, sup_e, nsup, hq_h, hrv_h, sidx_h, didx_h, gq_h, grv_h,
                  sidx, didx, bq, brv, gsem, oq_sem, orv_sem):
    w = lax.axis_index("s") * _NC + lax.axis_index("c")
    base = w * sup_e * nsup
    nin = sup_e // chk

    def sup(i, carry):
        off = base + i * sup_e
        pltpu.sync_copy(sidx_h.at[w, i], sidx)
        pltpu.sync_copy(didx_h.at[w, i], didx)
        for tab, idx, buf, out, osem in (
                (hq_h, sidx, bq, gq_h, oq_sem),
                (hrv_h, didx, brv, grv_h, orv_sem)):
            @pl.when(i > 0)
            def _wait_prev():
                pltpu.make_async_copy(buf, out.at[pl.ds(off, sup_e)],
                                      osem).wait()
            descs = [pltpu.async_copy(tab.at[idx.at[j]],
                                      buf.at[pl.ds(j * chk, chk)], gsem)
                     for j in range(nin)]
            for d in descs:
                d.wait()
            pltpu.async_copy(buf, out.at[pl.ds(off, sup_e)], osem)
        return carry

    lax.fori_loop(0, nsup, sup, 0)
    pltpu.make_async_copy(bq, gq_h.at[pl.ds(base, sup_e)], oq_sem).wait()
    pltpu.make_async_copy(brv, grv_h.at[pl.ds(base, sup_e)], orv_sem).wait()


def _gather2_call(hq, hrv, sidx4, didx4):
    nw, nsup, nin, chk = sidx4.shape
    n_e = int(np.prod(sidx4.shape))
    sup_e = nin * chk
    mesh = plsc.VectorSubcoreMesh(core_axis_name="c", subcore_axis_name="s")
    f = pl.kernel(
        functools.partial(_gather2_body, chk, sup_e, nsup),
        out_type=[jax.ShapeDtypeStruct((n_e, H), jnp.float32),
                  jax.ShapeDtypeStruct((n_e, H), jnp.int32)],
        mesh=mesh,
        scratch_types=[
            pltpu.VMEM((nin, chk), jnp.int32),
            pltpu.VMEM((nin, chk), jnp.int32),
            pltpu.VMEM((sup_e, H), jnp.float32),
            pltpu.VMEM((sup_e, H), jnp.int32),
            pltpu.SemaphoreType.DMA,
            pltpu.SemaphoreType.DMA,
            pltpu.SemaphoreType.DMA,
        ],
    )
    return f(hq, hrv, sidx4, didx4)


def _scatter_body(chk, rpw, msg_h, sidx_h, z_h, out_h, sidx, m0, m1, agg,
                  sem0, sem1):
    c = lax.axis_index("c")
    s = lax.axis_index("s")
    w = s * _NC + c
    base = w * rpw * chk
    zb = s * 624
    pltpu.sync_copy(z_h.at[pl.ds(zb, 640)], agg.at[pl.ds(zb, 640)])
    pltpu.sync_copy(sidx_h.at[w], sidx)
    plsc.subcore_barrier()

    pltpu.async_copy(msg_h.at[pl.ds(base, chk)], m0, sem0)
    pltpu.async_copy(msg_h.at[pl.ds(base + chk, chk)], m1, sem1)

    def body(i, carry):
        off = base + 2 * i * chk
        pltpu.make_async_copy(msg_h.at[pl.ds(off, chk)], m0, sem0).wait()
        pltpu.sync_copy(m0, agg.at[sidx.at[2 * i]], add=True)
        pltpu.async_copy(msg_h.at[pl.ds(off + 2 * chk, chk)], m0, sem0)
        pltpu.make_async_copy(msg_h.at[pl.ds(off + chk, chk)], m1,
                              sem1).wait()
        pltpu.sync_copy(m1, agg.at[sidx.at[2 * i + 1]], add=True)

        @pl.when(i < (rpw - 3) // 2)
        def _prefetch():
            pltpu.async_copy(msg_h.at[pl.ds(off + 3 * chk, chk)], m1, sem1)
        return carry

    lax.fori_loop(0, (rpw - 1) // 2, body, 0)
    pltpu.make_async_copy(msg_h.at[pl.ds(base + (rpw - 1) * chk, chk)],
                          m0, sem0).wait()
    pltpu.sync_copy(m0, agg.at[sidx.at[rpw - 1]], add=True)
    plsc.subcore_barrier()
    pltpu.sync_copy(agg.at[pl.ds(zb, 640)], out_h.at[c, pl.ds(zb, 640)])


def _scatter_call(msg, sidx3, zeros, n_nodes):
    nw, rpw, chk = sidx3.shape
    mesh = plsc.VectorSubcoreMesh(core_axis_name="c", subcore_axis_name="s")
    f = pl.kernel(
        functools.partial(_scatter_body, chk, rpw),
        out_type=jax.ShapeDtypeStruct((_NC, n_nodes, H), jnp.float32),
        mesh=mesh,
        scratch_types=[
            pltpu.VMEM((rpw, chk), jnp.int32),
            pltpu.VMEM((chk, H), jnp.float32),
            pltpu.VMEM((chk, H), jnp.float32),
            pltpu.VMEM_SHARED((n_nodes, H), jnp.float32),
            pltpu.SemaphoreType.DMA,
            pltpu.SemaphoreType.DMA,
        ],
    )
    return f(msg, sidx3, zeros)


def _inv_freq():
    half = H // 2
    dim_t = 10000.0 ** (2.0 * (np.arange(half) // 2).astype(np.float32) / half)
    inv = (1.0 / dim_t).astype(np.float32)
    return jnp.asarray(np.concatenate([inv, inv])[None, :])


def _t_vectors(t, params):
    half = H // 2
    freqs = jnp.exp(-np.log(10000.0) * jnp.arange(half, dtype=jnp.float32) / half)
    args = t[:, None] * freqs
    temb = jnp.concatenate([jnp.cos(args), jnp.sin(args)], axis=-1)
    temb = jax.nn.silu(temb @ params["tp1"]["W"] + params["tp1"]["b"])
    temb = temb @ params["tp2"]["W"] + params["tp2"]["b"]
    out = []
    for lp in params["layers"]:
        tv = jnp.maximum(temb @ lp["tmlp1"]["W"] + lp["tmlp1"]["b"], 0.0)
        tv = tv @ lp["tmlp2"]["W"] + lp["tmlp2"]["b"]
        out.append(tv)
    return out


def kernel(node_coords, edge_index, edge_distances, x_t, t, params):
    n_nodes = node_coords.shape[0]
    src, dst = edge_index[0], edge_index[1]

    n_edges = src.shape[0]
    invf = _inv_freq()
    h = _embed_call(node_coords[:, 0:1], node_coords[:, 1:2], invf)
    tvecs = _t_vectors(t, params)

    K = 2
    ec = n_edges // K
    perw = ec // _NW
    chk = next(b for b in (80, 40, 16, 8) if perw % b == 0)
    rpw = perw // chk
    nin = next(k for k in (5, 4, 2, 1) if rpw % k == 0)
    s4 = src.reshape(K, _NW, rpw // nin, nin, chk)
    d4 = dst.reshape(K, _NW, rpw // nin, nin, chk)
    s3 = src.reshape(K, _NW, rpw, chk)
    zeros = jnp.zeros((n_nodes, H), jnp.float32)

    n_layers = len(params["layers"])
    e = [None] * K
    outs = [None] * K
    for li, (lp, tv) in enumerate(zip(params["layers"], tvecs)):
        last = li == n_layers - 1
        wcat = jnp.concatenate([lp["Q"], lp["R"], lp["V"], lp["U"]], axis=1)
        hq, hrv, hu = _proj_call(h, wcat)
        evec = jnp.stack([lp["en_s"], lp["en_b"], lp["emlp1"]["b"],
                          lp["emlp2"]["b"] + tv[0]], axis=0)
        msg = [None] * K
        aggs = []
        for c in range(K):
            gq, grv = _gather2_call(hq, hrv, s4[c], d4[c])
            if li == 0:
                e[c], msg[c] = _edge_first_call(
                    edge_distances[c * ec:(c + 1) * ec, None],
                    x_t[c * ec:(c + 1) * ec, None], invf, gq, grv,
                    lp["P"], lp["emlp1"]["W"], lp["emlp2"]["W"], evec)
            elif last:
                outs[c] = _edge_last_call(
                    e[c], gq, grv, lp["P"], lp["emlp1"]["W"],
                    lp["emlp2"]["W"], evec, params["h1"]["W"],
                    params["h1"]["b"][None, :], params["h2"]["W"],
                    params["h2"]["b"][None, :])
            else:
                e[c], msg[c] = _edge_call(e[c], gq, grv, lp["P"],
                                          lp["emlp1"]["W"], lp["emlp2"]["W"],
                                          evec)
        if last:
            return jnp.concatenate(outs, axis=0)
        for c in range(K):
            parts = _scatter_call(msg[c], s3[c], zeros, n_nodes)
            aggs.extend([parts[0], parts[1]])
        nvec = jnp.stack([lp["nn_s"], lp["nn_b"]], axis=0)
        h = _nup_call(h, hu, aggs, nvec)


# speedup vs baseline: 1.0327x; 1.0327x over previous
import functools

import jax
import jax.numpy as jnp
import numpy as np
from jax import lax
from jax.experimental import pallas as pl
from jax.experimental.pallas import tpu as pltpu
from jax.experimental.pallas import tpu_sc as plsc

H = 128
_EPS = 1e-5

_NC, _NS = 2, 16
_NW = _NC * _NS
_CHK = 80
_RPW = 125
_SUP = 400
_NSUP = _RPW * _CHK // _SUP


def _blk(n, want):
    for b in range(min(want, n), 7, -1):
        if n % b == 0 and b % 8 == 0:
            return b
    return n


def _embed_body(a_ref, b_ref, invf_ref, out_ref):
    col = lax.broadcasted_iota(jnp.int32, (1, H), 1)
    val = jnp.where(col < (H // 2), a_ref[...], b_ref[...])
    arg = val * invf_ref[...]
    out_ref[...] = jnp.where(col % 2 == 0, jnp.sin(arg), jnp.cos(arg))


def _b16(x):
    u = jax.lax.bitcast_convert_type(x, jnp.uint32)
    return (u + jnp.uint32(0x7FFF) + ((u >> 16) & jnp.uint32(1))) >> 16


def _proj_body(h_ref, w_ref, o1_ref, o2_ref, o3_ref):
    hp = jnp.dot(h_ref[...], w_ref[...], preferred_element_type=jnp.float32)
    o1_ref[...] = hp[:, 0:H]
    packed = (_b16(hp[:, H:2 * H]) << 16) | _b16(hp[:, 2 * H:3 * H])
    o2_ref[...] = jax.lax.bitcast_convert_type(packed, jnp.int32)
    o3_ref[...] = hp[:, 3 * H:4 * H]


def _unpack_rv(grv):
    pk = jax.lax.bitcast_convert_type(grv, jnp.uint32)
    gr = jax.lax.bitcast_convert_type(pk & jnp.uint32(0xFFFF0000), jnp.float32)
    gv = jax.lax.bitcast_convert_type(pk << 16, jnp.float32)
    return gr, gv


def _edge_core(e, gq, gr, P_ref, W1_ref, W2_ref, vec_ref):
    ehat = jnp.dot(e, P_ref[...], preferred_element_type=jnp.float32)
    ehat = ehat + gq + gr
    m = jnp.mean(ehat, axis=-1, keepdims=True)
    v = jnp.mean((ehat - m) ** 2, axis=-1, keepdims=True)
    u = (ehat - m) * lax.rsqrt(v + _EPS) * vec_ref[0:1, :] + vec_ref[1:2, :]
    mid = jnp.maximum(
        jnp.dot(u, W1_ref[...], preferred_element_type=jnp.float32)
        + vec_ref[2:3, :], 0.0)
    eo = (e + jnp.dot(mid, W2_ref[...], preferred_element_type=jnp.float32)
          + vec_ref[3:4, :])
    return eo, ehat


def _edge_first_body(a_ref, b_ref, invf_ref, gq_ref, grv_ref, P_ref, W1_ref,
                     W2_ref, vec_ref, eo_ref, msg_ref):
    col = lax.broadcasted_iota(jnp.int32, (1, H), 1)
    val = jnp.where(col < (H // 2), a_ref[...], b_ref[...])
    arg = val * invf_ref[...]
    e = jnp.where(col % 2 == 0, jnp.sin(arg), jnp.cos(arg))
    gr, gv = _unpack_rv(grv_ref[...])
    eo, ehat = _edge_core(e, gq_ref[...], gr, P_ref, W1_ref, W2_ref, vec_ref)
    eo_ref[...] = eo
    msg_ref[...] = jax.nn.sigmoid(ehat) * gv


def _edge_body(e_ref, gq_ref, grv_ref, P_ref, W1_ref, W2_ref,
               vec_ref, eo_ref, msg_ref):
    gr, gv = _unpack_rv(grv_ref[...])
    eo, ehat = _edge_core(e_ref[...], gq_ref[...], gr, P_ref, W1_ref, W2_ref,
                          vec_ref)
    eo_ref[...] = eo
    msg_ref[...] = jax.nn.sigmoid(ehat) * gv


def _edge_last_body(e_ref, gq_ref, grv_ref, P_ref, W1_ref, W2_ref, vec_ref,
                    hW1_ref, hb1_ref, hW2_ref, hb2_ref, out_ref):
    gr, _ = _unpack_rv(grv_ref[...])
    eo, _ = _edge_core(e_ref[...], gq_ref[...], gr, P_ref, W1_ref, W2_ref,
                       vec_ref)
    mid = jnp.maximum(
        jnp.dot(eo, hW1_ref[...], preferred_element_type=jnp.float32)
        + hb1_ref[...], 0.0)
    out_ref[...] = (jnp.dot(mid, hW2_ref[...],
                            preferred_element_type=jnp.float32) + hb2_ref[...])


def _nup_body(h_ref, hu_ref, *rest):
    vec_ref, out_ref = rest[-2], rest[-1]
    x = hu_ref[...]
    for a_ref in rest[:-2]:
        x = x + a_ref[...]
    m = jnp.mean(x, axis=-1, keepdims=True)
    v = jnp.mean((x - m) ** 2, axis=-1, keepdims=True)
    u = (x - m) * lax.rsqrt(v + _EPS) * vec_ref[0:1, :] + vec_ref[1:2, :]
    out_ref[...] = h_ref[...] + jnp.maximum(u, 0.0)


def _nup_proj_body(h_ref, hu_ref, a0_ref, a1_ref, vec_ref, w_ref,
                   o1_ref, o2_ref, o3_ref, hn_ref):
    x = hu_ref[...] + a0_ref[...] + a1_ref[...]
    m = jnp.mean(x, axis=-1, keepdims=True)
    v = jnp.mean((x - m) ** 2, axis=-1, keepdims=True)
    u = (x - m) * lax.rsqrt(v + _EPS) * vec_ref[0:1, :] + vec_ref[1:2, :]
    hn = h_ref[...] + jnp.maximum(u, 0.0)
    hn_ref[...] = hn
    hp = jnp.dot(hn, w_ref[...], preferred_element_type=jnp.float32)
    o1_ref[...] = hp[:, 0:H]
    packed = (_b16(hp[:, H:2 * H]) << 16) | _b16(hp[:, 2 * H:3 * H])
    o2_ref[...] = jax.lax.bitcast_convert_type(packed, jnp.int32)
    o3_ref[...] = hp[:, 3 * H:4 * H]


def _full(shape):
    return pl.BlockSpec(shape, lambda *a: (0,) * len(shape))


def _embed_call(a, b, invf):
    n = a.shape[0]
    B = _blk(n, 4000)
    grid = (n // B,)
    return pl.pallas_call(
        _embed_body,
        grid=grid,
        in_specs=[pl.BlockSpec((B, 1), lambda i: (i, 0)),
                  pl.BlockSpec((B, 1), lambda i: (i, 0)),
                  _full((1, H))],
        out_specs=pl.BlockSpec((B, H), lambda i: (i, 0)),
        out_shape=jax.ShapeDtypeStruct((n, H), jnp.float32),
    )(a, b, invf)


def _proj_call(h, w):
    n, dout = h.shape[0], w.shape[1]
    return pl.pallas_call(
        _proj_body,
        in_specs=[_full((n, H)), _full((H, dout))],
        out_specs=[_full((n, H))] * 3,
        out_shape=[jax.ShapeDtypeStruct((n, H), jnp.float32),
                   jax.ShapeDtypeStruct((n, H), jnp.int32),
                   jax.ShapeDtypeStruct((n, H), jnp.float32)],
    )(h, w)


def _edge_call(e, gq, grv, P, W1, W2, vec):
    n = e.shape[0]
    B = _blk(n, 4000)
    grid = (n // B,)
    eb = pl.BlockSpec((B, H), lambda i: (i, 0))
    return pl.pallas_call(
        _edge_body,
        grid=grid,
        in_specs=[eb, eb, eb, _full((H, H)), _full((H, H)),
                  _full((H, H)), _full((4, H))],
        out_specs=[eb, eb],
        out_shape=[jax.ShapeDtypeStruct((n, H), jnp.float32),
                   jax.ShapeDtypeStruct((n, H), jnp.float32)],
    )(e, gq, grv, P, W1, W2, vec)


def _edge_first_call(a, b, invf, gq, grv, P, W1, W2, vec):
    n = gq.shape[0]
    B = _blk(n, 4000)
    grid = (n // B,)
    eb = pl.BlockSpec((B, H), lambda i: (i, 0))
    sb = pl.BlockSpec((B, 1), lambda i: (i, 0))
    return pl.pallas_call(
        _edge_first_body,
        grid=grid,
        in_specs=[sb, sb, _full((1, H)), eb, eb, _full((H, H)),
                  _full((H, H)), _full((H, H)), _full((4, H))],
        out_specs=[eb, eb],
        out_shape=[jax.ShapeDtypeStruct((n, H), jnp.float32),
                   jax.ShapeDtypeStruct((n, H), jnp.float32)],
    )(a, b, invf, gq, grv, P, W1, W2, vec)


def _edge_last_call(e, gq, grv, P, W1, W2, vec, hW1, hb1, hW2, hb2):
    n = e.shape[0]
    B = _blk(n, 4000)
    grid = (n // B,)
    eb = pl.BlockSpec((B, H), lambda i: (i, 0))
    dout = hW2.shape[1]
    return pl.pallas_call(
        _edge_last_body,
        grid=grid,
        in_specs=[eb, eb, eb, _full((H, H)), _full((H, H)), _full((H, H)),
                  _full((4, H)), _full((H, H)), _full((1, H)),
                  _full((H, dout)), _full((1, dout))],
        out_specs=pl.BlockSpec((B, dout), lambda i: (i, 0)),
        out_shape=jax.ShapeDtypeStruct((n, dout), jnp.float32),
    )(e, gq, grv, P, W1, W2, vec, hW1, hb1, hW2, hb2)


def _nup_call(h, hu, aggs, vec):
    n = h.shape[0]
    B = _blk(n, 4000)
    grid = (n // B,)
    nb = pl.BlockSpec((B, H), lambda i: (i, 0))
    return pl.pallas_call(
        _nup_body,
        grid=grid,
        in_specs=[nb, nb] + [nb] * len(aggs) + [_full((2, H))],
        out_specs=nb,
        out_shape=jax.ShapeDtypeStruct((n, H), jnp.float32),
    )(h, hu, *aggs, vec)


def _nup_proj_call(h, hu, a0, a1, vec, w):
    n = h.shape[0]
    return pl.pallas_call(
        _nup_proj_body,
        in_specs=[_full((n, H))] * 4 + [_full((2, H)), _full((H, 4 * H))],
        out_specs=[_full((n, H))] * 4,
        out_shape=[jax.ShapeDtypeStruct((n, H), jnp.float32),
                   jax.ShapeDtypeStruct((n, H), jnp.int32),
                   jax.ShapeDtypeStruct((n, H), jnp.float32),
                   jax.ShapeDtypeStruct((n, H), jnp.float32)],
    )(h, hu, a0, a1, vec, w)


def _gather2_body(chk, sup_e, nsup, hq_h, hrv_h, sidx_h, didx_h, gq_h, grv_h,
                  sidx, didx, bq, brv, gsem, oq_sem, orv_sem):
    w = lax.axis_index("s") * _NC + lax.axis_index("c")
    base = w * sup_e * nsup
    nin = sup_e // chk

    def sup(i, carry):
        off = base + i * sup_e
        pltpu.sync_copy(sidx_h.at[w, i], sidx)
        pltpu.sync_copy(didx_h.at[w, i], didx)
        for tab, idx, buf, out, osem in (
                (hq_h, sidx, bq, gq_h, oq_sem),
                (hrv_h, didx, brv, grv_h, orv_sem)):
            @pl.when(i > 0)
            def _wait_prev():
                pltpu.make_async_copy(buf, out.at[pl.ds(off, sup_e)],
                                      osem).wait()
            descs = [pltpu.async_copy(tab.at[idx.at[j]],
                                      buf.at[pl.ds(j * chk, chk)], gsem)
                     for j in range(nin)]
            for d in descs:
                d.wait()
            pltpu.async_copy(buf, out.at[pl.ds(off, sup_e)], osem)
        return carry

    lax.fori_loop(0, nsup, sup, 0)
    pltpu.make_async_copy(bq, gq_h.at[pl.ds(base, sup_e)], oq_sem).wait()
    pltpu.make_async_copy(brv, grv_h.at[pl.ds(base, sup_e)], orv_sem).wait()


def _gather2_call(hq, hrv, sidx4, didx4):
    nw, nsup, nin, chk = sidx4.shape
    n_e = int(np.prod(sidx4.shape))
    sup_e = nin * chk
    mesh = plsc.VectorSubcoreMesh(core_axis_name="c", subcore_axis_name="s")
    f = pl.kernel(
        functools.partial(_gather2_body, chk, sup_e, nsup),
        out_type=[jax.ShapeDtypeStruct((n_e, H), jnp.float32),
                  jax.ShapeDtypeStruct((n_e, H), jnp.int32)],
        mesh=mesh,
        scratch_types=[
            pltpu.VMEM((nin, chk), jnp.int32),
            pltpu.VMEM((nin, chk), jnp.int32),
            pltpu.VMEM((sup_e, H), jnp.float32),
            pltpu.VMEM((sup_e, H), jnp.int32),
            pltpu.SemaphoreType.DMA,
            pltpu.SemaphoreType.DMA,
            pltpu.SemaphoreType.DMA,
        ],
    )
    return f(hq, hrv, sidx4, didx4)


def _scatter_body(chk, rpw, msg_h, sidx_h, z_h, out_h, sidx, m0, m1, agg,
                  sem0, sem1):
    c = lax.axis_index("c")
    s = lax.axis_index("s")
    w = s * _NC + c
    base = w * rpw * chk
    zb = s * 624
    pltpu.sync_copy(z_h.at[pl.ds(zb, 640)], agg.at[pl.ds(zb, 640)])
    pltpu.sync_copy(sidx_h.at[w], sidx)
    plsc.subcore_barrier()

    pltpu.async_copy(msg_h.at[pl.ds(base, chk)], m0, sem0)
    pltpu.async_copy(msg_h.at[pl.ds(base + chk, chk)], m1, sem1)

    def body(i, carry):
        off = base + 2 * i * chk
        pltpu.make_async_copy(msg_h.at[pl.ds(off, chk)], m0, sem0).wait()
        pltpu.sync_copy(m0, agg.at[sidx.at[2 * i]], add=True)
        pltpu.async_copy(msg_h.at[pl.ds(off + 2 * chk, chk)], m0, sem0)
        pltpu.make_async_copy(msg_h.at[pl.ds(off + chk, chk)], m1,
                              sem1).wait()
        pltpu.sync_copy(m1, agg.at[sidx.at[2 * i + 1]], add=True)

        @pl.when(i < (rpw - 3) // 2)
        def _prefetch():
            pltpu.async_copy(msg_h.at[pl.ds(off + 3 * chk, chk)], m1, sem1)
        return carry

    lax.fori_loop(0, (rpw - 1) // 2, body, 0)
    pltpu.make_async_copy(msg_h.at[pl.ds(base + (rpw - 1) * chk, chk)],
                          m0, sem0).wait()
    pltpu.sync_copy(m0, agg.at[sidx.at[rpw - 1]], add=True)
    plsc.subcore_barrier()
    pltpu.sync_copy(agg.at[pl.ds(zb, 640)], out_h.at[c, pl.ds(zb, 640)])


def _scatter_call(msg, sidx3, zeros, n_nodes):
    nw, rpw, chk = sidx3.shape
    mesh = plsc.VectorSubcoreMesh(core_axis_name="c", subcore_axis_name="s")
    f = pl.kernel(
        functools.partial(_scatter_body, chk, rpw),
        out_type=jax.ShapeDtypeStruct((_NC, n_nodes, H), jnp.float32),
        mesh=mesh,
        scratch_types=[
            pltpu.VMEM((rpw, chk), jnp.int32),
            pltpu.VMEM((chk, H), jnp.float32),
            pltpu.VMEM((chk, H), jnp.float32),
            pltpu.VMEM_SHARED((n_nodes, H), jnp.float32),
            pltpu.SemaphoreType.DMA,
            pltpu.SemaphoreType.DMA,
        ],
    )
    return f(msg, sidx3, zeros)


def _inv_freq():
    half = H // 2
    dim_t = 10000.0 ** (2.0 * (np.arange(half) // 2).astype(np.float32) / half)
    inv = (1.0 / dim_t).astype(np.float32)
    return jnp.asarray(np.concatenate([inv, inv])[None, :])


def _t_vectors(t, params):
    half = H // 2
    freqs = jnp.exp(-np.log(10000.0) * jnp.arange(half, dtype=jnp.float32) / half)
    args = t[:, None] * freqs
    temb = jnp.concatenate([jnp.cos(args), jnp.sin(args)], axis=-1)
    temb = jax.nn.silu(temb @ params["tp1"]["W"] + params["tp1"]["b"])
    temb = temb @ params["tp2"]["W"] + params["tp2"]["b"]
    out = []
    for lp in params["layers"]:
        tv = jnp.maximum(temb @ lp["tmlp1"]["W"] + lp["tmlp1"]["b"], 0.0)
        tv = tv @ lp["tmlp2"]["W"] + lp["tmlp2"]["b"]
        out.append(tv)
    return out


def kernel(node_coords, edge_index, edge_distances, x_t, t, params):
    n_nodes = node_coords.shape[0]
    src, dst = edge_index[0], edge_index[1]

    n_edges = src.shape[0]
    invf = _inv_freq()
    h = _embed_call(node_coords[:, 0:1], node_coords[:, 1:2], invf)
    tvecs = _t_vectors(t, params)

    perw = n_edges // _NW
    chk = next(b for b in (80, 40, 16, 8) if perw % b == 0)
    rpw = perw // chk
    nin = next(k for k in (5, 4, 2, 1) if rpw % k == 0)
    s4 = src.reshape(_NW, rpw // nin, nin, chk)
    d4 = dst.reshape(_NW, rpw // nin, nin, chk)
    s3 = src.reshape(_NW, rpw, chk)
    zeros = jnp.zeros((n_nodes, H), jnp.float32)

    layers = params["layers"]
    n_layers = len(layers)
    wcat = [jnp.concatenate([lp["Q"], lp["R"], lp["V"], lp["U"]], axis=1)
            for lp in layers]
    hq, hrv, hu = _proj_call(h, wcat[0])
    e = None
    for li, (lp, tv) in enumerate(zip(layers, tvecs)):
        last = li == n_layers - 1
        gq, grv = _gather2_call(hq, hrv, s4, d4)
        evec = jnp.stack([lp["en_s"], lp["en_b"], lp["emlp1"]["b"],
                          lp["emlp2"]["b"] + tv[0]], axis=0)
        if li == 0:
            e, msg = _edge_first_call(
                edge_distances[:, None], x_t[:, None], invf, gq, grv,
                lp["P"], lp["emlp1"]["W"], lp["emlp2"]["W"], evec)
        elif last:
            return _edge_last_call(
                e, gq, grv, lp["P"], lp["emlp1"]["W"], lp["emlp2"]["W"],
                evec, params["h1"]["W"], params["h1"]["b"][None, :],
                params["h2"]["W"], params["h2"]["b"][None, :])
        else:
            e, msg = _edge_call(e, gq, grv, lp["P"], lp["emlp1"]["W"],
                                lp["emlp2"]["W"], evec)
        parts = _scatter_call(msg, s3, zeros, n_nodes)
        nvec = jnp.stack([lp["nn_s"], lp["nn_b"]], axis=0)
        hq, hrv, hu2, h = _nup_proj_call(h, hu, parts[0], parts[1], nvec,
                                         wcat[li + 1])
        hu = hu2
